# Initial kernel scaffold; baseline (speedup 1.0000x reference)
import jax
import jax.numpy as jnp
from jax.experimental import pallas as pl


def kernel(node_features, edge_index):
    raise NotImplementedError("write your pallas kernel here")


# speedup vs baseline: 4.3678x; 4.3678x over previous
import functools

import jax
import jax.numpy as jnp
from jax.experimental import pallas as pl
from jax.experimental.pallas import tpu as pltpu

N = 10000
E = 320000
D = 128
N_PAD = 10240


def _panel_body(at_ref, vt_ref, wt_ref, at_s, tt_s):
    k, n = at_ref.shape
    lanes_n = jax.lax.broadcasted_iota(jnp.int32, (1, n), 1)
    lanes_k = jax.lax.broadcasted_iota(jnp.int32, (1, k), 1)
    rows_k = jax.lax.broadcasted_iota(jnp.int32, (k, 1), 0)

    at_s[...] = at_ref[...]
    vt_ref[...] = jnp.zeros_like(vt_ref)
    tt_s[...] = jnp.zeros_like(tt_s)

    def step(j, carry):
        x = at_s[pl.ds(j, 1), :]
        sel = lanes_n == j
        tail = lanes_n >= j
        xm = jnp.where(tail, x, 0.0)
        alpha = jnp.sum(jnp.where(sel, x, 0.0))
        norm2 = jnp.sum(xm * xm)
        norm = jnp.sqrt(norm2)
        beta = jnp.where(alpha >= 0, -norm, norm)
        denom = alpha - beta
        safe_denom = jnp.where(denom == 0, 1.0, denom)
        safe_beta = jnp.where(beta == 0, 1.0, beta)
        tau = jnp.where(norm2 > 0, (beta - alpha) / safe_beta, 0.0)
        v = jnp.where(sel, 1.0, xm / safe_denom)
        vt_ref[pl.ds(j, 1), :] = v

        w = jax.lax.dot_general(at_s[...], v, (((1,), (1,)), ((), ())),
                                preferred_element_type=jnp.float32)
        wm = jnp.where(rows_k > j, w, 0.0)
        at_s[...] = at_s[...] - tau * wm * v

        u = jax.lax.dot_general(v, vt_ref[...], (((1,), (1,)), ((), ())),
                                preferred_element_type=jnp.float32)
        um = jnp.where(lanes_k < j, u, 0.0)
        trow = -tau * jax.lax.dot_general(um, tt_s[...], (((1,), (0,)), ((), ())),
                                          preferred_element_type=jnp.float32)
        tt_s[pl.ds(j, 1), :] = trow + jnp.where(lanes_k == j, tau, 0.0)
        return carry

    jax.lax.fori_loop(0, k, step, 0, unroll=False)
    wt_ref[...] = jax.lax.dot_general(tt_s[...], vt_ref[...],
                                      (((1,), (0,)), ((), ())),
                                      preferred_element_type=jnp.float32)


def _panel(at_pad):
    k = at_pad.shape[0]
    return pl.pallas_call(
        _panel_body,
        out_shape=(
            jax.ShapeDtypeStruct((k, N_PAD), jnp.float32),
            jax.ShapeDtypeStruct((k, N_PAD), jnp.float32),
        ),
        scratch_shapes=[
            pltpu.VMEM((k, N_PAD), jnp.float32),
            pltpu.VMEM((k, k), jnp.float32),
        ],
    )(at_pad)


def _q_body(wt_ref, vt_ref, q_ref, *, bm, bn):
    i = pl.program_id(0)
    j = pl.program_id(1)
    prod = jax.lax.dot_general(wt_ref[...], vt_ref[...],
                               (((0,), (0,)), ((), ())),
                               preferred_element_type=jnp.float32)
    r = i * bm + jax.lax.broadcasted_iota(jnp.int32, (bm, bn), 0)
    c = j * bn + jax.lax.broadcasted_iota(jnp.int32, (bm, bn), 1)
    q_ref[...] = jnp.where(r == c, 1.0, 0.0) - prod


def _form_q(vt, wt, bm=512, bn=512):
    k = vt.shape[0]
    grid = (pl.cdiv(N, bm), pl.cdiv(N, bn))
    return pl.pallas_call(
        functools.partial(_q_body, bm=bm, bn=bn),
        grid=grid,
        in_specs=[
            pl.BlockSpec((k, bm), lambda i, j: (0, i)),
            pl.BlockSpec((k, bn), lambda i, j: (0, j)),
        ],
        out_specs=pl.BlockSpec((bm, bn), lambda i, j: (i, j)),
        out_shape=jax.ShapeDtypeStruct((N, N), jnp.float32),
    )(wt, vt)


def kernel(node_features, edge_index):
    src = edge_index[:, 0]
    dst = edge_index[:, 1]

    ones = jnp.ones((E,), dtype=jnp.float32)
    deg = jax.ops.segment_sum(ones, src, num_segments=N)
    dinv = jnp.power(deg, -0.5)
    y = dinv[:, None] * node_features
    z = jax.ops.segment_sum(jnp.take(y, dst, axis=0), src, num_segments=N)
    seg_mean = jnp.where(deg[:, None] > 0, deg[:, None] * node_features, 0.0)
    agg = 0.5 * dinv[:, None] * z + 0.5 * seg_mean

    at_pad = jnp.pad(agg.T, ((0, 0), (0, N_PAD - N)))
    vt, wt = _panel(at_pad)
    return _form_q(vt, wt)
